# baseline (speedup 1.0000x reference)
import functools

import jax
import jax.numpy as jnp
from jax import lax
from jax.experimental import pallas as pl
from jax.experimental.pallas import tpu as pltpu
from jax.experimental.pallas import tpu_sc as plsc

NC = 2
NS = 16
NW = NC * NS
CHUNK = 128
NBUF = 5


def _make_gather(V, D, B):
    assert B % (NW * CHUNK) == 0
    bpw = B // NW
    nch = bpw // CHUNK
    assert nch % NBUF == 0
    mesh = plsc.VectorSubcoreMesh(
        core_axis_name="c", subcore_axis_name="s",
        num_cores=NC, num_subcores=NS)

    @functools.partial(
        pl.kernel,
        out_type=jax.ShapeDtypeStruct((B, D), jnp.float32),
        mesh=mesh,
        scratch_types=[
            pltpu.VMEM((nch, CHUNK), jnp.int32),
            [pltpu.VMEM((CHUNK, D), jnp.float32)] * NBUF,
            [pltpu.SemaphoreType.DMA] * NBUF,
            [pltpu.SemaphoreType.DMA] * NBUF,
        ],
    )
    def gather_kernel(table_hbm, idx_hbm, out_hbm, idx_v, bufs, gsems, ssems):
        wid = lax.axis_index("s") * NC + lax.axis_index("c")
        base = wid * bpw
        pltpu.sync_copy(idx_hbm.at[wid], idx_v)

        def out_slice(j):
            return out_hbm.at[pl.ds(base + j * CHUNK, CHUNK)]

        for b in range(NBUF):
            pltpu.async_copy(table_hbm.at[pl.ds(b * CHUNK, CHUNK)], bufs[b], gsems[b])

        @pl.loop(0, nch // NBUF)
        def _(g):
            j0 = g * NBUF
            for b in range(NBUF):
                pltpu.make_async_copy(
                    table_hbm.at[pl.ds((j0 + b) * CHUNK, CHUNK)], bufs[b], gsems[b]).wait()
                pltpu.async_copy(bufs[b], out_slice(j0 + b), ssems[b])
            for b in range(NBUF):
                @pl.when(j0 + b + NBUF < nch)
                def _():
                    pltpu.make_async_copy(
                        bufs[b], out_slice(j0 + b), ssems[b]).wait()
                    pltpu.async_copy(
                        table_hbm.at[pl.ds((j0 + b + NBUF) * CHUNK, CHUNK)],
                        bufs[b], gsems[b])

        for b in range(NBUF):
            pltpu.make_async_copy(
                bufs[b], out_slice(nch - NBUF + b), ssems[b]).wait()

    return gather_kernel


def kernel(x, table):
    V, D = table.shape
    S, T = x.shape
    B = x.size
    idx = x.T.reshape(NW, B // (NW * CHUNK), CHUNK).astype(jnp.int32)
    out = _make_gather(V, D, B)(table, idx)
    return out.reshape(T, S, D).transpose(1, 0, 2)


# speedup vs baseline: 1.2576x; 1.2576x over previous
import functools

import jax
import jax.numpy as jnp
from jax import lax
from jax.experimental import pallas as pl
from jax.experimental.pallas import tpu as pltpu
from jax.experimental.pallas import tpu_sc as plsc

NC = 2
NS = 16
NW = NC * NS
CHUNK = 128
NBUF = 5


def _make_gather(V, D, B):
    assert B % (NW * CHUNK) == 0
    bpw = B // NW
    nch = bpw // CHUNK
    assert nch % NBUF == 0
    mesh = plsc.VectorSubcoreMesh(
        core_axis_name="c", subcore_axis_name="s",
        num_cores=NC, num_subcores=NS)

    @functools.partial(
        pl.kernel,
        out_type=jax.ShapeDtypeStruct((B, D), jnp.float32),
        mesh=mesh,
        scratch_types=[
            pltpu.VMEM((nch, CHUNK), jnp.int32),
            pltpu.VMEM((nch, CHUNK), jnp.int32),
            [pltpu.VMEM((CHUNK, D), jnp.float32)] * NBUF,
            [pltpu.SemaphoreType.DMA] * NBUF,
            [pltpu.SemaphoreType.DMA] * NBUF,
        ],
    )
    def gather_kernel(table_hbm, idx_hbm, oidx_hbm, out_hbm, idx_v, oidx_v, bufs, gsems, ssems):
        wid = lax.axis_index("s") * NC + lax.axis_index("c")
        base = wid * bpw
        pltpu.sync_copy(idx_hbm.at[wid], idx_v)
        pltpu.sync_copy(oidx_hbm, oidx_v)

        def out_slice(j):
            return out_hbm.at[pl.ds(base, bpw)].at[oidx_v.at[j]]

        for b in range(NBUF):
            pltpu.async_copy(table_hbm.at[idx_v.at[b]], bufs[b], gsems[b])

        @pl.loop(0, nch // NBUF)
        def _(g):
            j0 = g * NBUF
            for b in range(NBUF):
                pltpu.make_async_copy(
                    table_hbm.at[idx_v.at[j0 + b]], bufs[b], gsems[b]).wait()
                pltpu.async_copy(bufs[b], out_slice(j0 + b), ssems[b])
            for b in range(NBUF):
                @pl.when(j0 + b + NBUF < nch)
                def _():
                    pltpu.make_async_copy(
                        bufs[b], out_slice(j0 + b), ssems[b]).wait()
                    pltpu.async_copy(
                        table_hbm.at[idx_v.at[j0 + b + NBUF]],
                        bufs[b], gsems[b])

        for b in range(NBUF):
            pltpu.make_async_copy(
                bufs[b], out_slice(nch - NBUF + b), ssems[b]).wait()

    return gather_kernel


def kernel(x, table):
    V, D = table.shape
    S, T = x.shape
    B = x.size
    idx = x.T.reshape(NW, B // (NW * CHUNK), CHUNK).astype(jnp.int32)
    oidx = jnp.arange(B // NW, dtype=jnp.int32).reshape(-1, CHUNK)
    out = _make_gather(V, D, B)(table, idx, oidx)
    return out.reshape(T, S, D).transpose(1, 0, 2)


# speedup vs baseline: 1.2766x; 1.0151x over previous
import functools

import jax
import jax.numpy as jnp
from jax import lax
from jax.experimental import pallas as pl
from jax.experimental.pallas import tpu as pltpu
from jax.experimental.pallas import tpu_sc as plsc

NC = 2
NS = 16
NW = NC * NS
CHUNK = 128
NBUF = 5


def _make_gather(V, D, B):
    assert B % (NW * CHUNK) == 0
    bpw = B // NW
    nch = bpw // CHUNK
    assert nch % NBUF == 0
    mesh = plsc.VectorSubcoreMesh(
        core_axis_name="c", subcore_axis_name="s",
        num_cores=NC, num_subcores=NS)

    @functools.partial(
        pl.kernel,
        out_type=jax.ShapeDtypeStruct((B, D), jnp.float32),
        mesh=mesh,
        scratch_types=[
            pltpu.VMEM((nch, CHUNK), jnp.int32),
            [pltpu.VMEM((CHUNK, D), jnp.float32)] * NBUF,
            [pltpu.SemaphoreType.DMA] * NBUF,
            [pltpu.SemaphoreType.DMA] * NBUF,
        ],
    )
    def gather_kernel(table_hbm, idx_hbm, out_hbm, idx_v, bufs, gsems, ssems):
        wid = lax.axis_index("s") * NC + lax.axis_index("c")
        base = wid * bpw
        pltpu.sync_copy(idx_hbm.at[wid], idx_v)

        def out_slice(j):
            return out_hbm.at[pl.ds(base + j * CHUNK, CHUNK)]

        for b in range(NBUF):
            pltpu.async_copy(table_hbm.at[idx_v.at[b]], bufs[b], gsems[b])

        @pl.loop(0, nch // NBUF)
        def _(g):
            j0 = g * NBUF
            for b in range(NBUF):
                pltpu.make_async_copy(
                    table_hbm.at[idx_v.at[j0 + b]], bufs[b], gsems[b]).wait()
                pltpu.async_copy(bufs[b], out_slice(j0 + b), ssems[b])
            for b in range(NBUF):
                @pl.when(j0 + b + NBUF < nch)
                def _():
                    pltpu.make_async_copy(
                        bufs[b], out_slice(j0 + b), ssems[b]).wait()
                    pltpu.async_copy(
                        table_hbm.at[idx_v.at[j0 + b + NBUF]],
                        bufs[b], gsems[b])

        for b in range(NBUF):
            pltpu.make_async_copy(
                bufs[b], out_slice(nch - NBUF + b), ssems[b]).wait()

    return gather_kernel


def kernel(x, table):
    V, D = table.shape
    S, T = x.shape
    B = x.size
    idx = x.T.reshape(NW, B // (NW * CHUNK), CHUNK).astype(jnp.int32)
    out = _make_gather(V, D, B)(table, idx)
    return out.reshape(T, S, D).transpose(1, 0, 2)
